# Initial kernel scaffold; baseline (speedup 1.0000x reference)
import jax
import jax.numpy as jnp
from jax.experimental import pallas as pl


def kernel(x, edge_index, batch, W1, b1, W2, b2, W3, b3, W4, b4):
    raise NotImplementedError("write your pallas kernel here")


# speedup vs baseline: 9.3706x; 9.3706x over previous
import jax
import jax.numpy as jnp
from jax import lax
from jax.experimental import pallas as pl
from jax.experimental.pallas import tpu as pltpu
from jax.experimental.pallas import tpu_sc as plsc

N = 10000
E = 320000
D = 128
G = 16

NC, NS, LANES = 2, 16, 16
NW = NC * NS
ET = E // NW
K = 80
CH = ET // K
NP = 10112
ROWS = NP // NS
DH = D // 2


def _deg_body(dst_hbm, out_hbm, idx_d, ones_b, zbuf, acc, sem):
    c = lax.axis_index("c")
    s = lax.axis_index("s")
    wid = s * NC + c

    def zinit(i, _):
        zbuf[i, :] = jnp.zeros((LANES,), jnp.float32)
        return 0
    lax.fori_loop(0, ROWS, zinit, 0)

    def oinit(i, _):
        ones_b[i, :] = jnp.ones((LANES,), jnp.float32)
        return 0
    lax.fori_loop(0, K, oinit, 0)

    pltpu.sync_copy(zbuf, acc.at[pl.ds(s * ROWS, ROWS)])
    plsc.subcore_barrier()

    def step(ci, _):
        b = wid * ET + ci * K
        pltpu.sync_copy(dst_hbm.at[pl.ds(b, K)], idx_d)
        pltpu.sync_copy(ones_b, acc.at[idx_d], add=True)
        return 0
    lax.fori_loop(0, CH, step, 0)

    plsc.subcore_barrier()
    pltpu.sync_copy(acc.at[pl.ds(s * ROWS, ROWS)],
                    out_hbm.at[pl.ds(c * NP + s * ROWS, ROWS)])


_deg_call = pl.kernel(
    _deg_body,
    out_type=jax.ShapeDtypeStruct((NC * NP, LANES), jnp.float32),
    mesh=plsc.VectorSubcoreMesh(core_axis_name="c", subcore_axis_name="s"),
    compiler_params=pltpu.CompilerParams(use_tc_tiling_on_sc=False),
    scratch_types=[
        pltpu.VMEM((K,), jnp.int32),
        pltpu.VMEM((K, LANES), jnp.float32),
        pltpu.VMEM((ROWS, LANES), jnp.float32),
        pltpu.VMEM_SHARED((NP, LANES), jnp.float32),
        pltpu.SemaphoreType.DMA,
    ],
)


def _agg_body(vals_hbm, src_hbm, dst_hbm, out_hbm,
              idx_s, idx_d, rows, zbuf, acc, sem):
    c = lax.axis_index("c")
    s = lax.axis_index("s")
    wid = s * NC + c

    def zinit(i, _):
        r = i // (DH // LANES)
        q = i % (DH // LANES)
        zbuf[r, pl.ds(q * LANES, LANES)] = jnp.zeros((LANES,), jnp.float32)
        return 0
    lax.fori_loop(0, ROWS * (DH // LANES), zinit, 0)

    for h in (0, 1):
        pltpu.sync_copy(zbuf, acc.at[pl.ds(s * ROWS, ROWS)])
        plsc.subcore_barrier()

        def step(ci, _):
            b = wid * ET + ci * K
            pltpu.sync_copy(src_hbm.at[pl.ds(b, K)], idx_s)
            for j in range(K // LANES):
                v = idx_s[pl.ds(j * LANES, LANES)]
                idx_s[pl.ds(j * LANES, LANES)] = v * 2 + h
            pltpu.async_copy(vals_hbm.at[idx_s], rows, sem).wait()
            pltpu.sync_copy(dst_hbm.at[pl.ds(b, K)], idx_d)
            pltpu.sync_copy(rows, acc.at[idx_d], add=True)
            return 0
        lax.fori_loop(0, CH, step, 0)

        plsc.subcore_barrier()
        pltpu.sync_copy(acc.at[pl.ds(s * ROWS, ROWS)],
                        out_hbm.at[h, pl.ds(c * NP + s * ROWS, ROWS)])
        plsc.subcore_barrier()


_agg_call = pl.kernel(
    _agg_body,
    out_type=jax.ShapeDtypeStruct((2, NC * NP, DH), jnp.float32),
    mesh=plsc.VectorSubcoreMesh(core_axis_name="c", subcore_axis_name="s"),
    compiler_params=pltpu.CompilerParams(use_tc_tiling_on_sc=False),
    scratch_types=[
        pltpu.VMEM((K,), jnp.int32),
        pltpu.VMEM((K,), jnp.int32),
        pltpu.VMEM((K, DH), jnp.float32),
        pltpu.VMEM((ROWS, DH), jnp.float32),
        pltpu.VMEM_SHARED((NP, DH), jnp.float32),
        pltpu.SemaphoreType.DMA,
    ],
)


def _agg(vals, src, dst):
    vals2 = vals.reshape(2 * N, DH)
    return _agg_call(vals2, src, dst).reshape(2, NC, NP, DH)


def _sum_parts(p_ref):
    lo = p_ref[0, 0] + p_ref[0, 1]
    hi = p_ref[1, 0] + p_ref[1, 1]
    return jnp.concatenate([lo, hi], axis=1)


R = 1000


def _tc_scale_body(degp_ref, x_ref, xs_ref, dinv_ref):
    deg = degp_ref[0] + degp_ref[1] + 1.0
    dinv = lax.rsqrt(jnp.maximum(deg, 1.0))
    d2 = dinv[:, 0:1]
    xs_ref[...] = x_ref[...] * d2
    dinv_ref[...] = d2


def _tc_scale(degp, x):
    grid = N // R
    return pl.pallas_call(
        _tc_scale_body,
        grid=(grid,),
        in_specs=[
            pl.BlockSpec((2, R, LANES), lambda i: (0, i, 0)),
            pl.BlockSpec((R, D), lambda i: (i, 0)),
        ],
        out_specs=[
            pl.BlockSpec((R, D), lambda i: (i, 0)),
            pl.BlockSpec((R, 1), lambda i: (i, 0)),
        ],
        out_shape=[
            jax.ShapeDtypeStruct((N, D), jnp.float32),
            jax.ShapeDtypeStruct((N, 1), jnp.float32),
        ],
    )(degp, x)


def _tc_layer1_body(pa_ref, xs_ref, dinv_ref, W1_ref, b1_ref, W2_ref, ms_ref):
    t = (_sum_parts(pa_ref) + xs_ref[...]) * dinv_ref[...]
    h = jnp.dot(t, W1_ref[...], preferred_element_type=jnp.float32)
    h = jnp.maximum(h + b1_ref[...], 0.0)
    m = jnp.dot(h, W2_ref[...], preferred_element_type=jnp.float32)
    ms_ref[...] = m * dinv_ref[...]


def _tc_layer1(pa, xs, dinv, W1, b1, W2):
    grid = N // R
    hid = W1.shape[1]
    return pl.pallas_call(
        _tc_layer1_body,
        grid=(grid,),
        in_specs=[
            pl.BlockSpec((2, NC, R, DH), lambda i: (0, 0, i, 0)),
            pl.BlockSpec((R, D), lambda i: (i, 0)),
            pl.BlockSpec((R, 1), lambda i: (i, 0)),
            pl.BlockSpec((D, hid), lambda i: (0, 0)),
            pl.BlockSpec((1, hid), lambda i: (0, 0)),
            pl.BlockSpec((hid, D), lambda i: (0, 0)),
        ],
        out_specs=pl.BlockSpec((R, D), lambda i: (i, 0)),
        out_shape=jax.ShapeDtypeStruct((N, D), jnp.float32),
    )(pa, xs, dinv, W1, b1, W2)


def _tc_final_body(pb_ref, ms_ref, dinv_ref, batch_ref, b2_ref,
                   W3_ref, b3_ref, W4_ref, b4_ref,
                   xh_ref, z_ref, gemb_ref, pooled, counts):
    i = pl.program_id(0)
    z = (_sum_parts(pb_ref) + ms_ref[...]) * dinv_ref[...] + b2_ref[...]
    z_ref[...] = z
    d = jnp.dot(z, W3_ref[...], preferred_element_type=jnp.float32)
    d = jnp.maximum(d + b3_ref[...], 0.0)
    xh_ref[...] = jnp.dot(d, W4_ref[...],
                          preferred_element_type=jnp.float32) + b4_ref[...]

    gids = lax.broadcasted_iota(jnp.int32, (R, G), 1)
    oh = (batch_ref[...] == gids).astype(jnp.float32)

    @pl.when(i == 0)
    def _():
        pooled[...] = jnp.zeros_like(pooled)
        counts[...] = jnp.zeros_like(counts)

    dn = (((0,), (0,)), ((), ()))
    pooled[...] += lax.dot_general(oh, z, dn,
                                   preferred_element_type=jnp.float32)
    counts[...] += lax.dot_general(oh, jnp.ones((R, D), jnp.float32), dn,
                                   preferred_element_type=jnp.float32)
    gemb_ref[...] = pooled[...] / jnp.maximum(counts[...], 1.0)


def _tc_final(pb, ms, dinv, batch2, b2, W3, b3, W4, b4):
    grid = N // R
    hid = W3.shape[1]
    return pl.pallas_call(
        _tc_final_body,
        grid=(grid,),
        in_specs=[
            pl.BlockSpec((2, NC, R, DH), lambda i: (0, 0, i, 0)),
            pl.BlockSpec((R, D), lambda i: (i, 0)),
            pl.BlockSpec((R, 1), lambda i: (i, 0)),
            pl.BlockSpec((R, 1), lambda i: (i, 0)),
            pl.BlockSpec((1, D), lambda i: (0, 0)),
            pl.BlockSpec((D, hid), lambda i: (0, 0)),
            pl.BlockSpec((1, hid), lambda i: (0, 0)),
            pl.BlockSpec((hid, D), lambda i: (0, 0)),
            pl.BlockSpec((1, D), lambda i: (0, 0)),
        ],
        out_specs=[
            pl.BlockSpec((R, D), lambda i: (i, 0)),
            pl.BlockSpec((R, D), lambda i: (i, 0)),
            pl.BlockSpec((G, D), lambda i: (0, 0)),
        ],
        out_shape=[
            jax.ShapeDtypeStruct((N, D), jnp.float32),
            jax.ShapeDtypeStruct((N, D), jnp.float32),
            jax.ShapeDtypeStruct((G, D), jnp.float32),
        ],
        scratch_shapes=[
            pltpu.VMEM((G, D), jnp.float32),
            pltpu.VMEM((G, D), jnp.float32),
        ],
    )(pb, ms, dinv, batch2, b2, W3, b3, W4, b4)


def kernel(x, edge_index, batch, W1, b1, W2, b2, W3, b3, W4, b4):
    src = edge_index[0]
    dst = edge_index[1]
    batch2 = batch[:, None]

    degp = _deg_call(dst).reshape(2, NP, LANES)
    xs, dinv = _tc_scale(degp, x)

    def body(i, carry):
        _, v, _ = carry
        p = _agg(v, src, dst)
        v_new = _tc_layer1(p, v, dinv, W1, b1[None, :], W2)
        return (v, v_new, p)

    p0 = jnp.zeros((2, NC, NP, DH), jnp.float32)
    niter = 2 + (src[0] < -1).astype(jnp.int32)
    ms, _, pb = lax.fori_loop(0, niter, body, (xs, xs, p0))

    x_hat, z, gemb = _tc_final(pb, ms, dinv, batch2, b2[None, :],
                               W3, b3[None, :], W4, b4[None, :])
    return (x_hat, z, gemb)
